# baseline (speedup 1.0000x reference)
import functools

import jax
import jax.numpy as jnp
from jax import lax
from jax.experimental import pallas as pl
from jax.experimental.pallas import tpu as pltpu
from jax.experimental.pallas import tpu_sc as plsc

N = 10000
E = 320000
D = 128
DH = 64

N_PAD = 10240
CHUNK = 128
H_CHUNKS = 80
P_CHUNKS = 160
NBUF = 5
GLA = 3
E_PAD = 16 * P_CHUNKS * CHUNK
ROWS_PER_SUBCORE = N_PAD // 16

_mesh = plsc.VectorSubcoreMesh(core_axis_name="c", subcore_axis_name="s")


@functools.partial(
    pl.kernel,
    out_type=jax.ShapeDtypeStruct((2, N_PAD, 16), jnp.float32),
    mesh=_mesh,
    scratch_types=[
        pltpu.VMEM((H_CHUNKS, CHUNK), jnp.int32),
        pltpu.VMEM((CHUNK, 16), jnp.float32),
        pltpu.VMEM_SHARED((N_PAD, 16), jnp.float32),
        pltpu.SemaphoreType.DMA,
    ],
)
def _histo_sc(dst_hbm, out_hbm, idx_v, ones_v, acc_sh, hsem):
    c = lax.axis_index("c")
    s = lax.axis_index("s")
    wid = c * 16 + s
    base = s * ROWS_PER_SUBCORE

    @pl.loop(0, CHUNK)
    def _(i):
        ones_v[i, :] = jnp.zeros((16,), jnp.float32)

    @pl.loop(0, ROWS_PER_SUBCORE // CHUNK)
    def _(k):
        pltpu.sync_copy(ones_v, acc_sh.at[pl.ds(base + k * CHUNK, CHUNK)])

    @pl.loop(0, CHUNK)
    def _(i):
        ones_v[i, :] = jnp.ones((16,), jnp.float32)

    pltpu.sync_copy(dst_hbm.at[wid], idx_v)
    plsc.subcore_barrier()

    K = 8
    for j in range(K):
        pltpu.async_copy(ones_v, acc_sh.at[idx_v.at[j]], hsem, add=True)

    @pl.loop(0, H_CHUNKS - K)
    def _(j):
        pltpu.make_async_copy(ones_v, acc_sh.at[idx_v.at[j]], hsem).wait()
        pltpu.async_copy(ones_v, acc_sh.at[idx_v.at[j + K]], hsem, add=True)

    for j in range(H_CHUNKS - K, H_CHUNKS):
        pltpu.make_async_copy(ones_v, acc_sh.at[idx_v.at[j]], hsem).wait()

    plsc.subcore_barrier()
    pltpu.sync_copy(
        acc_sh.at[pl.ds(base, ROWS_PER_SUBCORE)],
        out_hbm.at[c, pl.ds(base, ROWS_PER_SUBCORE)],
    )


@functools.partial(
    pl.kernel,
    out_type=jax.ShapeDtypeStruct((2, N_PAD, DH), jnp.float32),
    mesh=_mesh,
    scratch_types=[
        pltpu.VMEM((P_CHUNKS, CHUNK), jnp.int32),
        pltpu.VMEM((P_CHUNKS, CHUNK), jnp.int32),
        pltpu.VMEM((NBUF, CHUNK, DH), jnp.float32),
        pltpu.VMEM_SHARED((N_PAD, DH), jnp.float32),
    ] + [pltpu.SemaphoreType.DMA] * (2 * NBUF),
    compiler_params=pltpu.CompilerParams(use_tc_tiling_on_sc=False),
)
def _prop_sc(g_hbm, src_hbm, dst_hbm, out_hbm, src_v, dst_v, rows_v, acc_sh,
             *sems):
    c = lax.axis_index("c")
    s = lax.axis_index("s")
    base = s * ROWS_PER_SUBCORE
    gc = g_hbm.at[c]

    @pl.loop(0, CHUNK)
    def _(i):
        @pl.loop(0, DH // 16)
        def _(l):
            rows_v[0, i, pl.ds(l * 16, 16)] = jnp.zeros((16,), jnp.float32)

    @pl.loop(0, ROWS_PER_SUBCORE // CHUNK)
    def _(k):
        pltpu.sync_copy(rows_v.at[0], acc_sh.at[pl.ds(base + k * CHUNK, CHUNK)])

    pltpu.sync_copy(src_hbm.at[s], src_v)
    pltpu.sync_copy(dst_hbm.at[s], dst_v)
    plsc.subcore_barrier()

    gsems = sems[:NBUF]
    ssems = sems[NBUF:]
    SLAG = NBUF - GLA

    def _ig(b, j):
        pltpu.async_copy(gc.at[src_v.at[j]], rows_v.at[b], gsems[b])

    def _wg(b, j):
        pltpu.make_async_copy(gc.at[src_v.at[j]], rows_v.at[b], gsems[b]).wait()

    def _is(b, j):
        pltpu.async_copy(rows_v.at[b], acc_sh.at[dst_v.at[j]], ssems[b],
                         add=True)

    def _ws(b, j):
        pltpu.make_async_copy(rows_v.at[b], acc_sh.at[dst_v.at[j]],
                              ssems[b]).wait()

    for j in range(GLA):
        _ig(j % NBUF, j)
    for j in range(SLAG):
        _ig((j + GLA) % NBUF, j + GLA)
        _wg(j % NBUF, j)
        _is(j % NBUF, j)

    @pl.loop(0, (P_CHUNKS - NBUF) // NBUF)
    def _(grp):
        for k in range(NBUF):
            j = SLAG + grp * NBUF + k
            b = (SLAG + k) % NBUF
            bg = (SLAG + k + GLA) % NBUF
            _ws(bg, j - SLAG)
            _ig(bg, j + GLA)
            _wg(b, j)
            _is(b, j)

    for k in range(GLA):
        j = P_CHUNKS - GLA + k
        _wg(j % NBUF, j)
        _is(j % NBUF, j)
    for j in range(P_CHUNKS - NBUF, P_CHUNKS):
        _ws(j % NBUF, j)

    plsc.subcore_barrier()
    pltpu.sync_copy(
        acc_sh.at[pl.ds(base, ROWS_PER_SUBCORE)],
        out_hbm.at[c, pl.ds(base, ROWS_PER_SUBCORE)],
    )


_BLK = 512
_GRID = N_PAD // _BLK


def _dinv_block(c0_ref, c1_ref, pid):
    cnt = c0_ref[:, :1] + c1_ref[:, :1] + 1.0
    rows = lax.broadcasted_iota(jnp.int32, (_BLK, 1), 0) + pid * _BLK
    return jnp.where(rows < N, lax.rsqrt(cnt), 0.0)


def _split_store(o_ref, p):
    o_ref[0] = p[:, :DH]
    o_ref[1] = p[:, DH:]


def _mm1_body(x_ref, w_ref, c0_ref, c1_ref, o_ref):
    dinv = _dinv_block(c0_ref, c1_ref, pl.program_id(0))
    p = jnp.dot(x_ref[...], w_ref[...], preferred_element_type=jnp.float32)
    _split_store(o_ref, p * dinv)


def _mm2_body(a_ref, g_ref, b_ref, w_ref, c0_ref, c1_ref, o_ref):
    dinv = _dinv_block(c0_ref, c1_ref, pl.program_id(0))
    agg = jnp.concatenate([a_ref[0] + g_ref[0], a_ref[1] + g_ref[1]], axis=1)
    h = jnp.maximum(dinv * agg + b_ref[...], 0.0)
    p = jnp.dot(h, w_ref[...], preferred_element_type=jnp.float32)
    _split_store(o_ref, p * dinv)


def _fin_body(a_ref, g_ref, b_ref, c0_ref, c1_ref, o_ref):
    dinv = _dinv_block(c0_ref, c1_ref, pl.program_id(0))
    agg = jnp.concatenate([a_ref[0] + g_ref[0], a_ref[1] + g_ref[1]], axis=1)
    o_ref[...] = dinv * agg + b_ref[...]


def _row_spec():
    return pl.BlockSpec((_BLK, D), lambda i: (i, 0))


def _half_spec():
    return pl.BlockSpec((2, _BLK, DH), lambda i: (0, i, 0))


def _cnt_spec():
    return pl.BlockSpec((_BLK, 16), lambda i: (i, 0))


def _full_spec(shape):
    return pl.BlockSpec(shape, lambda i: (0,) * len(shape))


_HALF_TY = jax.ShapeDtypeStruct((2, N_PAD, DH), jnp.float32)


def _mm1(x_pad, W1, cnt0, cnt1):
    return pl.pallas_call(
        _mm1_body,
        grid=(_GRID,),
        in_specs=[_row_spec(), _full_spec((D, D)), _cnt_spec(), _cnt_spec()],
        out_specs=_half_spec(),
        out_shape=_HALF_TY,
    )(x_pad, W1, cnt0, cnt1)


def _mm2(acc1, g1, b1, W2, cnt0, cnt1):
    return pl.pallas_call(
        _mm2_body,
        grid=(_GRID,),
        in_specs=[_half_spec(), _half_spec(), _full_spec((1, D)),
                  _full_spec((D, D)), _cnt_spec(), _cnt_spec()],
        out_specs=_half_spec(),
        out_shape=_HALF_TY,
    )(acc1, g1, b1, W2, cnt0, cnt1)


def _fin(acc2, g2, b2, cnt0, cnt1):
    return pl.pallas_call(
        _fin_body,
        grid=(_GRID,),
        in_specs=[_half_spec(), _half_spec(), _full_spec((1, D)),
                  _cnt_spec(), _cnt_spec()],
        out_specs=_row_spec(),
        out_shape=jax.ShapeDtypeStruct((N_PAD, D), jnp.float32),
    )(acc2, g2, b2, cnt0, cnt1)


def kernel(x, edge_index, W1, b1, W2, b2):
    pad = E_PAD - E
    src = jnp.concatenate([edge_index[0], jnp.full((pad,), N, jnp.int32)])
    dst = jnp.concatenate([edge_index[1], jnp.full((pad,), N, jnp.int32)])
    src_p = src.reshape(16, P_CHUNKS, CHUNK)
    dst_p = dst.reshape(16, P_CHUNKS, CHUNK)
    dst_h = dst.reshape(32, H_CHUNKS, CHUNK)
    x_pad = jnp.pad(x, ((0, N_PAD - N), (0, 0)))
    b1r = b1.reshape(1, D)
    b2r = b2.reshape(1, D)

    cnt = _histo_sc(dst_h)
    cnt0, cnt1 = cnt[0], cnt[1]

    g1 = _mm1(x_pad, W1, cnt0, cnt1)
    acc1 = _prop_sc(g1, src_p, dst_p)
    g2 = _mm2(acc1, g1, b1r, W2, cnt0, cnt1)
    acc2 = _prop_sc(g2, src_p, dst_p)
    out = _fin(acc2, g2, b2r, cnt0, cnt1)
    return out[:N]


# speedup vs baseline: 1.0887x; 1.0887x over previous
import functools

import jax
import jax.numpy as jnp
from jax import lax
from jax.experimental import pallas as pl
from jax.experimental.pallas import tpu as pltpu
from jax.experimental.pallas import tpu_sc as plsc

N = 10000
E = 320000
D = 128
DH = 64

N_PAD = 10240
CHUNK = 128
H_CHUNKS = 80
P_CHUNKS = 160
NBUF = 5
GLA = 3
E_PAD = 16 * P_CHUNKS * CHUNK
ROWS_PER_SUBCORE = N_PAD // 16

_mesh = plsc.VectorSubcoreMesh(core_axis_name="c", subcore_axis_name="s")


@functools.partial(
    pl.kernel,
    out_type=jax.ShapeDtypeStruct((2, N_PAD, 16), jnp.float32),
    mesh=_mesh,
    scratch_types=[
        pltpu.VMEM((H_CHUNKS, CHUNK), jnp.int32),
        pltpu.VMEM((CHUNK, 16), jnp.float32),
        pltpu.VMEM_SHARED((N_PAD, 16), jnp.float32),
        pltpu.SemaphoreType.DMA,
    ],
)
def _histo_sc(dst_hbm, out_hbm, idx_v, ones_v, acc_sh, hsem):
    c = lax.axis_index("c")
    s = lax.axis_index("s")
    wid = c * 16 + s
    base = s * ROWS_PER_SUBCORE

    @pl.loop(0, CHUNK)
    def _(i):
        ones_v[i, :] = jnp.zeros((16,), jnp.float32)

    @pl.loop(0, ROWS_PER_SUBCORE // CHUNK)
    def _(k):
        pltpu.sync_copy(ones_v, acc_sh.at[pl.ds(base + k * CHUNK, CHUNK)])

    @pl.loop(0, CHUNK)
    def _(i):
        ones_v[i, :] = jnp.ones((16,), jnp.float32)

    pltpu.sync_copy(dst_hbm.at[wid], idx_v)
    plsc.subcore_barrier()

    K = 8
    for j in range(K):
        pltpu.async_copy(ones_v, acc_sh.at[idx_v.at[j]], hsem, add=True)

    @pl.loop(0, H_CHUNKS - K)
    def _(j):
        pltpu.make_async_copy(ones_v, acc_sh.at[idx_v.at[j]], hsem).wait()
        pltpu.async_copy(ones_v, acc_sh.at[idx_v.at[j + K]], hsem, add=True)

    for j in range(H_CHUNKS - K, H_CHUNKS):
        pltpu.make_async_copy(ones_v, acc_sh.at[idx_v.at[j]], hsem).wait()

    plsc.subcore_barrier()
    pltpu.sync_copy(
        acc_sh.at[pl.ds(base, ROWS_PER_SUBCORE)],
        out_hbm.at[c, pl.ds(base, ROWS_PER_SUBCORE)],
    )


@functools.partial(
    pl.kernel,
    out_type=jax.ShapeDtypeStruct((2, N_PAD, DH), jnp.float32),
    mesh=_mesh,
    scratch_types=[
        pltpu.VMEM((P_CHUNKS, CHUNK), jnp.int32),
        pltpu.VMEM((P_CHUNKS, CHUNK), jnp.int32),
        pltpu.VMEM((NBUF, CHUNK, DH), jnp.float32),
        pltpu.VMEM_SHARED((N_PAD, DH), jnp.float32),
    ] + [pltpu.SemaphoreType.DMA] * (2 * NBUF),
    compiler_params=pltpu.CompilerParams(use_tc_tiling_on_sc=False),
)
def _prop_sc(g_hbm, src_hbm, dst_hbm, out_hbm, src_v, dst_v, rows_v, acc_sh,
             *sems):
    c = lax.axis_index("c")
    s = lax.axis_index("s")
    base = s * ROWS_PER_SUBCORE
    gc = g_hbm.at[c]

    @pl.loop(0, CHUNK)
    def _(i):
        @pl.loop(0, DH // 16)
        def _(l):
            rows_v[0, i, pl.ds(l * 16, 16)] = jnp.zeros((16,), jnp.float32)

    @pl.loop(0, ROWS_PER_SUBCORE // CHUNK)
    def _(k):
        pltpu.sync_copy(rows_v.at[0], acc_sh.at[pl.ds(base + k * CHUNK, CHUNK)])

    pltpu.sync_copy(src_hbm.at[s], src_v)
    pltpu.sync_copy(dst_hbm.at[s], dst_v)
    plsc.subcore_barrier()

    gsems = sems[:NBUF]
    ssems = sems[NBUF:]
    SLAG = NBUF - GLA

    def _ig(b, j):
        pltpu.async_copy(gc.at[src_v.at[j]], rows_v.at[b], gsems[b])

    def _wg(b, j):
        pltpu.make_async_copy(gc.at[src_v.at[j]], rows_v.at[b], gsems[b]).wait()

    def _is(b, j):
        pltpu.async_copy(rows_v.at[b], acc_sh.at[dst_v.at[j]], ssems[b],
                         add=True)

    def _ws(b, j):
        pltpu.make_async_copy(rows_v.at[b], acc_sh.at[dst_v.at[j]],
                              ssems[b]).wait()

    for j in range(GLA):
        _ig(j % NBUF, j)
    for j in range(SLAG):
        _ig((j + GLA) % NBUF, j + GLA)
        _wg(j % NBUF, j)
        _is(j % NBUF, j)

    @pl.loop(0, (P_CHUNKS - NBUF) // NBUF)
    def _(grp):
        for k in range(NBUF):
            j = SLAG + grp * NBUF + k
            b = (SLAG + k) % NBUF
            bg = (SLAG + k + GLA) % NBUF
            _ws(bg, j - SLAG)
            _ig(bg, j + GLA)
            _wg(b, j)
            _is(b, j)

    for k in range(GLA):
        j = P_CHUNKS - GLA + k
        _wg(j % NBUF, j)
        _is(j % NBUF, j)
    for j in range(P_CHUNKS - NBUF, P_CHUNKS):
        _ws(j % NBUF, j)

    plsc.subcore_barrier()
    pltpu.sync_copy(
        acc_sh.at[pl.ds(base, ROWS_PER_SUBCORE)],
        out_hbm.at[c, pl.ds(base, ROWS_PER_SUBCORE)],
    )


_BLK = 512
_GRID = N_PAD // _BLK


def _dinv_block(c0_ref, c1_ref, pid):
    cnt = c0_ref[:, :1] + c1_ref[:, :1] + 1.0
    rows = lax.broadcasted_iota(jnp.int32, (_BLK, 1), 0) + pid * _BLK
    return jnp.where(rows < N, lax.rsqrt(cnt), 0.0)


def _split_store(o_ref, p):
    o_ref[0] = p[:, :DH]
    o_ref[1] = p[:, DH:]


def _p1_body(x_ref, w_ref, o_ref):
    o_ref[...] = jnp.dot(x_ref[...], w_ref[...],
                         preferred_element_type=jnp.float32)


def _scale1_body(p_ref, c0_ref, c1_ref, o_ref):
    dinv = _dinv_block(c0_ref, c1_ref, pl.program_id(0))
    _split_store(o_ref, p_ref[...] * dinv)


def _mm2_body(a_ref, g_ref, b_ref, w_ref, c0_ref, c1_ref, o_ref):
    dinv = _dinv_block(c0_ref, c1_ref, pl.program_id(0))
    agg = jnp.concatenate([a_ref[0] + g_ref[0], a_ref[1] + g_ref[1]], axis=1)
    h = jnp.maximum(dinv * agg + b_ref[...], 0.0)
    p = jnp.dot(h, w_ref[...], preferred_element_type=jnp.float32)
    _split_store(o_ref, p * dinv)


def _fin_body(a_ref, g_ref, b_ref, c0_ref, c1_ref, o_ref):
    dinv = _dinv_block(c0_ref, c1_ref, pl.program_id(0))
    agg = jnp.concatenate([a_ref[0] + g_ref[0], a_ref[1] + g_ref[1]], axis=1)
    o_ref[...] = dinv * agg + b_ref[...]


def _row_spec():
    return pl.BlockSpec((_BLK, D), lambda i: (i, 0))


def _half_spec():
    return pl.BlockSpec((2, _BLK, DH), lambda i: (0, i, 0))


def _cnt_spec():
    return pl.BlockSpec((_BLK, 16), lambda i: (i, 0))


def _full_spec(shape):
    return pl.BlockSpec(shape, lambda i: (0,) * len(shape))


_HALF_TY = jax.ShapeDtypeStruct((2, N_PAD, DH), jnp.float32)


def _p1(x_pad, W1):
    return pl.pallas_call(
        _p1_body,
        grid=(_GRID,),
        in_specs=[_row_spec(), _full_spec((D, D))],
        out_specs=_row_spec(),
        out_shape=jax.ShapeDtypeStruct((N_PAD, D), jnp.float32),
    )(x_pad, W1)


def _scale1(p1, cnt0, cnt1):
    return pl.pallas_call(
        _scale1_body,
        grid=(_GRID,),
        in_specs=[_row_spec(), _cnt_spec(), _cnt_spec()],
        out_specs=_half_spec(),
        out_shape=_HALF_TY,
    )(p1, cnt0, cnt1)


def _mm2(acc1, g1, b1, W2, cnt0, cnt1):
    return pl.pallas_call(
        _mm2_body,
        grid=(_GRID,),
        in_specs=[_half_spec(), _half_spec(), _full_spec((1, D)),
                  _full_spec((D, D)), _cnt_spec(), _cnt_spec()],
        out_specs=_half_spec(),
        out_shape=_HALF_TY,
    )(acc1, g1, b1, W2, cnt0, cnt1)


def _fin(acc2, g2, b2, cnt0, cnt1):
    return pl.pallas_call(
        _fin_body,
        grid=(_GRID,),
        in_specs=[_half_spec(), _half_spec(), _full_spec((1, D)),
                  _cnt_spec(), _cnt_spec()],
        out_specs=_row_spec(),
        out_shape=jax.ShapeDtypeStruct((N_PAD, D), jnp.float32),
    )(acc2, g2, b2, cnt0, cnt1)


def kernel(x, edge_index, W1, b1, W2, b2):
    pad = E_PAD - E
    src = jnp.concatenate([edge_index[0], jnp.full((pad,), N, jnp.int32)])
    dst = jnp.concatenate([edge_index[1], jnp.full((pad,), N, jnp.int32)])
    src_p = src.reshape(16, P_CHUNKS, CHUNK)
    dst_p = dst.reshape(16, P_CHUNKS, CHUNK)
    dst_h = dst.reshape(32, H_CHUNKS, CHUNK)
    x_pad = jnp.pad(x, ((0, N_PAD - N), (0, 0)))
    b1r = b1.reshape(1, D)
    b2r = b2.reshape(1, D)

    p1 = _p1(x_pad, W1)
    cnt = _histo_sc(dst_h)
    cnt0, cnt1 = cnt[0], cnt[1]

    g1 = _scale1(p1, cnt0, cnt1)
    acc1 = _prop_sc(g1, src_p, dst_p)
    g2 = _mm2(acc1, g1, b1r, W2, cnt0, cnt1)
    acc2 = _prop_sc(g2, src_p, dst_p)
    out = _fin(acc2, g2, b2r, cnt0, cnt1)
    return out[:N]
